# Initial kernel scaffold; baseline (speedup 1.0000x reference)
import jax
import jax.numpy as jnp
from jax.experimental import pallas as pl


def kernel(x, edge_index, W1, b1, W2, b2, eps):
    raise NotImplementedError("write your pallas kernel here")


# speedup vs baseline: 3.5128x; 3.5128x over previous
import jax
import jax.numpy as jnp
from jax import lax
from jax.experimental import pallas as pl
from jax.experimental.pallas import tpu as pltpu
from jax.experimental.pallas import tpu_sc as plsc

N = 10000
D = 128
E = 320000

NC = 2
NS = 16
W = 128
IB = 16
NB = 5
NCH = NB * IB
E_PAD = NC * NS * NCH * W
PAD_N = 10112
STRIPE = PAD_N // NS
OUT_ROWS = 624


def _sc_agg_body(x_hbm, row_hbm, col_hbm, z_hbm, out_hbm,
                 rb0, rb1, cb0, cb1, g0, g1, agg_sh, semA, semB, semI):
    c = lax.axis_index("c")
    s = lax.axis_index("s")

    pltpu.sync_copy(z_hbm, agg_sh.at[pl.ds(s * STRIPE, STRIPE)])

    base = (c * NS + s) * NCH

    def start_idx(b, rb, cb):
        pltpu.async_copy(row_hbm.at[pl.ds(base + b * IB, IB)], rb, semI)
        pltpu.async_copy(col_hbm.at[pl.ds(base + b * IB, IB)], cb, semI)

    def wait_idx(b, rb, cb):
        pltpu.make_async_copy(row_hbm.at[pl.ds(base + b * IB, IB)], rb,
                              semI).wait()
        pltpu.make_async_copy(col_hbm.at[pl.ds(base + b * IB, IB)], cb,
                              semI).wait()

    start_idx(0, rb0, cb0)

    plsc.subcore_barrier()

    idx_bufs = [(rb0, cb0), (rb1, cb1)]
    for b in range(NB):
        rb, cb = idx_bufs[b % 2]
        wait_idx(b, rb, cb)
        if b + 1 < NB:
            start_idx(b + 1, *idx_bufs[(b + 1) % 2])

        @pl.loop(0, IB)
        def _(j):
            @pl.loop(0, W, step=16)
            def _(k):
                r = rb[j, pl.ds(k, 16)]
                cc = cb[j, pl.ds(k, 16)]
                rb[j, pl.ds(k, 16)] = jnp.where(
                    r == cc, jnp.int32(N) + (r & jnp.int32(63)), r)

        pltpu.async_copy(x_hbm.at[cb.at[0]], g0, semA)
        pltpu.async_copy(x_hbm.at[cb.at[1]], g1, semB)

        @pl.loop(0, IB - 2, step=2)
        def _(j):
            pltpu.make_async_copy(x_hbm.at[cb.at[j]], g0, semA).wait()
            pltpu.sync_copy(g0, agg_sh.at[rb.at[j]], add=True)
            pltpu.async_copy(x_hbm.at[cb.at[j + 2]], g0, semA)
            pltpu.make_async_copy(x_hbm.at[cb.at[j + 1]], g1, semB).wait()
            pltpu.sync_copy(g1, agg_sh.at[rb.at[j + 1]], add=True)
            pltpu.async_copy(x_hbm.at[cb.at[j + 3]], g1, semB)

        pltpu.make_async_copy(x_hbm.at[cb.at[IB - 2]], g0, semA).wait()
        pltpu.sync_copy(g0, agg_sh.at[rb.at[IB - 2]], add=True)
        pltpu.make_async_copy(x_hbm.at[cb.at[IB - 1]], g1, semB).wait()
        pltpu.sync_copy(g1, agg_sh.at[rb.at[IB - 1]], add=True)

    plsc.subcore_barrier()
    pltpu.sync_copy(agg_sh.at[pl.ds(s * OUT_ROWS, OUT_ROWS)],
                    out_hbm.at[c, pl.ds(s * OUT_ROWS, OUT_ROWS)])

    @pl.when(s == 0)
    def _():
        tail = NS * OUT_ROWS
        pltpu.sync_copy(agg_sh.at[pl.ds(tail, N - tail)],
                        out_hbm.at[c, pl.ds(tail, N - tail)])


def _sc_aggregate(x, row_p, col_p, zeros):
    mesh = plsc.VectorSubcoreMesh(core_axis_name="c", subcore_axis_name="s")
    f = pl.kernel(
        _sc_agg_body,
        out_type=jax.ShapeDtypeStruct((NC, N, D), jnp.float32),
        mesh=mesh,
        scratch_types=[
            pltpu.VMEM((IB, W), jnp.int32),
            pltpu.VMEM((IB, W), jnp.int32),
            pltpu.VMEM((IB, W), jnp.int32),
            pltpu.VMEM((IB, W), jnp.int32),
            pltpu.VMEM((W, D), jnp.float32),
            pltpu.VMEM((W, D), jnp.float32),
            pltpu.VMEM_SHARED((PAD_N, D), jnp.float32),
            pltpu.SemaphoreType.DMA,
            pltpu.SemaphoreType.DMA,
            pltpu.SemaphoreType.DMA,
        ],
    )
    return f(x, row_p, col_p, zeros)


def _mlp_body(x_ref, agg_ref, w1_ref, b1_ref, w2_ref, b2_ref, eps_ref, o_ref):
    out = (x_ref[...] * (1.0 + eps_ref[0])
           + agg_ref[0] + agg_ref[1])
    h = jnp.dot(out, w1_ref[...], preferred_element_type=jnp.float32)
    h = jnp.maximum(h + b1_ref[...], 0.0)
    o_ref[...] = (jnp.dot(h, w2_ref[...], preferred_element_type=jnp.float32)
                  + b2_ref[...])


def _mlp(x, agg, W1, b1, W2, b2, eps):
    R = 1000
    grid = (N // R,)
    return pl.pallas_call(
        _mlp_body,
        grid=grid,
        in_specs=[
            pl.BlockSpec((R, D), lambda i: (i, 0)),
            pl.BlockSpec((NC, R, D), lambda i: (0, i, 0)),
            pl.BlockSpec((D, D), lambda i: (0, 0)),
            pl.BlockSpec((1, D), lambda i: (0, 0)),
            pl.BlockSpec((D, D), lambda i: (0, 0)),
            pl.BlockSpec((1, D), lambda i: (0, 0)),
            pl.BlockSpec(memory_space=pltpu.SMEM),
        ],
        out_specs=pl.BlockSpec((R, D), lambda i: (i, 0)),
        out_shape=jax.ShapeDtypeStruct((N, D), jnp.float32),
    )(x, agg, W1, b1.reshape(1, D), W2, b2.reshape(1, D), eps)


def kernel(x, edge_index, W1, b1, W2, b2, eps):
    row = edge_index[0]
    col = edge_index[1]
    npad = E_PAD - E
    pad_row = (N + (jnp.arange(npad, dtype=jnp.int32) % (PAD_N - N)))
    row_p = jnp.concatenate([row, pad_row]).reshape(E_PAD // W, W)
    col_p = jnp.concatenate(
        [col, jnp.zeros((npad,), jnp.int32)]).reshape(E_PAD // W, W)
    zeros = jnp.zeros((STRIPE, D), jnp.float32)
    agg = _sc_aggregate(x, row_p, col_p, zeros)
    return _mlp(x, agg, W1, b1, W2, b2, eps)


# speedup vs baseline: 10.7091x; 3.0486x over previous
import jax
import jax.numpy as jnp
from jax import lax
from jax.experimental import pallas as pl
from jax.experimental.pallas import tpu as pltpu
from jax.experimental.pallas import tpu_sc as plsc

N = 10000
D = 128
E = 320000

NC = 2
NS = 16
W = 128
NCH = 80
E_PAD = NC * NS * NCH * W
PAD_N = 10112
STRIPE = PAD_N // NS
OUT_ROWS = 624
SHIFT = 14
MASK = (1 << SHIFT) - 1


def _sc_agg_body(x_hbm, pk_hbm, z_hbm, out_hbm,
                 pk_v, colr, rowr, g0, g1, agg_sh, sg0, sg1, ss0, ss1):
    c = lax.axis_index("c")
    s = lax.axis_index("s")

    pltpu.sync_copy(z_hbm, agg_sh.at[pl.ds(s * STRIPE, STRIPE)])

    base = (c * NS + s) * NCH
    pltpu.sync_copy(pk_hbm.at[pl.ds(base, NCH)], pk_v)

    def unpack(m, ring):
        @pl.loop(0, W, step=16)
        def _(k):
            p = pk_v[m, pl.ds(k, 16)]
            cc = p & jnp.int32(MASK)
            r = jax.lax.shift_right_logical(p, SHIFT)
            colr[ring, pl.ds(k, 16)] = cc
            rowr[ring, pl.ds(k, 16)] = jnp.where(
                r == cc, jnp.int32(N) + (r & jnp.int32(63)), r)

    gbufs = (g0, g1)
    gsems = (sg0, sg1)
    ssems = (ss0, ss1)

    def start_gather(j, b):
        pltpu.async_copy(x_hbm.at[colr.at[b]], gbufs[b], gsems[b])

    def wait_gather(j, b):
        pltpu.make_async_copy(x_hbm.at[colr.at[b]], gbufs[b],
                              gsems[b]).wait()

    def start_scatter(j, b):
        pltpu.async_copy(gbufs[b], agg_sh.at[rowr.at[b]], ssems[b],
                         add=True)

    def wait_scatter(j, b):
        pltpu.make_async_copy(gbufs[b], agg_sh.at[rowr.at[b]],
                              ssems[b]).wait()

    unpack(0, 0)
    unpack(1, 1)

    plsc.subcore_barrier()

    start_gather(0, 0)
    start_gather(1, 1)

    @pl.loop(0, NCH - 2, step=2)
    def _(j):
        wait_gather(j, 0)
        start_scatter(j, 0)
        wait_gather(j + 1, 1)
        start_scatter(j + 1, 1)
        wait_scatter(j, 0)
        unpack(j + 2, 0)
        start_gather(j + 2, 0)
        wait_scatter(j + 1, 1)
        unpack(j + 3, 1)
        start_gather(j + 3, 1)

    wait_gather(NCH - 2, 0)
    start_scatter(NCH - 2, 0)
    wait_gather(NCH - 1, 1)
    start_scatter(NCH - 1, 1)
    wait_scatter(NCH - 2, 0)
    wait_scatter(NCH - 1, 1)

    plsc.subcore_barrier()
    pltpu.sync_copy(agg_sh.at[pl.ds(s * OUT_ROWS, OUT_ROWS)],
                    out_hbm.at[c, pl.ds(s * OUT_ROWS, OUT_ROWS)])

    @pl.when(s == 0)
    def _():
        tail = NS * OUT_ROWS
        pltpu.sync_copy(agg_sh.at[pl.ds(tail, N - tail)],
                        out_hbm.at[c, pl.ds(tail, N - tail)])


def _sc_aggregate(x, packed, zeros):
    mesh = plsc.VectorSubcoreMesh(core_axis_name="c", subcore_axis_name="s")
    f = pl.kernel(
        _sc_agg_body,
        out_type=jax.ShapeDtypeStruct((NC, N, D), jnp.float32),
        mesh=mesh,
        scratch_types=[
            pltpu.VMEM((NCH, W), jnp.int32),
            pltpu.VMEM((2, W), jnp.int32),
            pltpu.VMEM((2, W), jnp.int32),
            pltpu.VMEM((W, D), jnp.float32),
            pltpu.VMEM((W, D), jnp.float32),
            pltpu.VMEM_SHARED((PAD_N, D), jnp.float32),
        ] + [pltpu.SemaphoreType.DMA] * 4,
    )
    return f(x, packed, zeros)


def _mlp_body(x_ref, agg_ref, w1_ref, b1_ref, w2_ref, b2_ref, eps_ref, o_ref):
    out = (x_ref[...] * (1.0 + eps_ref[0])
           + agg_ref[0] + agg_ref[1])
    h = jnp.dot(out, w1_ref[...], preferred_element_type=jnp.float32)
    h = jnp.maximum(h + b1_ref[...], 0.0)
    o_ref[...] = (jnp.dot(h, w2_ref[...], preferred_element_type=jnp.float32)
                  + b2_ref[...])


def _mlp(x, agg, W1, b1, W2, b2, eps):
    R = 1000
    grid = (N // R,)
    return pl.pallas_call(
        _mlp_body,
        grid=grid,
        in_specs=[
            pl.BlockSpec((R, D), lambda i: (i, 0)),
            pl.BlockSpec((NC, R, D), lambda i: (0, i, 0)),
            pl.BlockSpec((D, D), lambda i: (0, 0)),
            pl.BlockSpec((1, D), lambda i: (0, 0)),
            pl.BlockSpec((D, D), lambda i: (0, 0)),
            pl.BlockSpec((1, D), lambda i: (0, 0)),
            pl.BlockSpec(memory_space=pltpu.SMEM),
        ],
        out_specs=pl.BlockSpec((R, D), lambda i: (i, 0)),
        out_shape=jax.ShapeDtypeStruct((N, D), jnp.float32),
    )(x, agg, W1, b1.reshape(1, D), W2, b2.reshape(1, D), eps)


def kernel(x, edge_index, W1, b1, W2, b2, eps):
    row = edge_index[0]
    col = edge_index[1]
    npad = E_PAD - E
    ar = jnp.arange(npad, dtype=jnp.int32)
    pad_row = N + ar % (PAD_N - N)
    pad_col = ar % N
    row_p = jnp.concatenate([row, pad_row])
    col_p = jnp.concatenate([col, pad_col])
    packed = ((row_p << SHIFT) | col_p).reshape(E_PAD // W, W)
    zeros = jnp.zeros((STRIPE, D), jnp.float32)
    agg = _sc_aggregate(x, packed, zeros)
    return _mlp(x, agg, W1, b1, W2, b2, eps)


# speedup vs baseline: 13.7529x; 1.2842x over previous
import jax
import jax.numpy as jnp
from jax import lax
from jax.experimental import pallas as pl
from jax.experimental.pallas import tpu as pltpu
from jax.experimental.pallas import tpu_sc as plsc

N = 10000
D = 128
E = 320000

NC = 2
NS = 16
W = 64
NCH = 160
E_PAD = NC * NS * NCH * W
PAD_N = 10112
STRIPE = PAD_N // NS
OUT_ROWS = 624
SHIFT = 14
MASK = (1 << SHIFT) - 1


def _sc_agg_body(x_hbm, pk_hbm, z_hbm, out_hbm,
                 pk_v, colr, rowr, g0, g1, g2, g3, agg_sh,
                 sg0, sg1, sg2, sg3, ss0, ss1, ss2, ss3):
    c = lax.axis_index("c")
    s = lax.axis_index("s")

    pltpu.sync_copy(z_hbm, agg_sh.at[pl.ds(s * STRIPE, STRIPE)])

    base = (c * NS + s) * (NCH // 2)
    pltpu.sync_copy(pk_hbm.at[pl.ds(base, NCH // 2)], pk_v)

    def unpack(m, ring):
        @pl.loop(0, W, step=16)
        def _(k):
            p = pk_v[m >> 1, pl.ds((m & 1) * 64 + k, 16)]
            cc = p & jnp.int32(MASK)
            r = jax.lax.shift_right_logical(p, SHIFT)
            colr[ring, pl.ds(k, 16)] = cc
            rowr[ring, pl.ds(k, 16)] = jnp.where(
                r == cc, jnp.int32(N) + (r & jnp.int32(63)), r)

    gbufs = (g0, g1, g2, g3)
    gsems = (sg0, sg1, sg2, sg3)
    ssems = (ss0, ss1, ss2, ss3)

    def start_gather(j, b):
        pltpu.async_copy(x_hbm.at[colr.at[b]], gbufs[b], gsems[b])

    def wait_gather(j, b):
        pltpu.make_async_copy(x_hbm.at[colr.at[b]], gbufs[b],
                              gsems[b]).wait()

    def start_scatter(j, b):
        pltpu.async_copy(gbufs[b], agg_sh.at[rowr.at[b]], ssems[b],
                         add=True)

    def wait_scatter(j, b):
        pltpu.make_async_copy(gbufs[b], agg_sh.at[rowr.at[b]],
                              ssems[b]).wait()

    unpack(0, 0)
    unpack(1, 1)

    plsc.subcore_barrier()

    start_gather(0, 0)
    start_gather(1, 1)
    wait_gather(0, 0)
    start_scatter(0, 0)
    unpack(2, 2)
    start_gather(2, 2)
    wait_gather(1, 1)
    start_scatter(1, 1)
    unpack(3, 3)
    start_gather(3, 3)

    @pl.loop(2, NCH - 2, step=4)
    def _(j):
        for k in range(4):
            wait_scatter(j + k - 2, k)
            unpack(j + k + 2, k)
            start_gather(j + k + 2, k)
            wait_gather(j + k, (2 + k) % 4)
            start_scatter(j + k, (2 + k) % 4)

    wait_gather(NCH - 2, (NCH - 2) % 4)
    start_scatter(NCH - 2, (NCH - 2) % 4)
    wait_gather(NCH - 1, (NCH - 1) % 4)
    start_scatter(NCH - 1, (NCH - 1) % 4)
    for m in range(NCH - 4, NCH):
        wait_scatter(m, m % 4)

    plsc.subcore_barrier()
    pltpu.sync_copy(agg_sh.at[pl.ds(s * OUT_ROWS, OUT_ROWS)],
                    out_hbm.at[c, pl.ds(s * OUT_ROWS, OUT_ROWS)])

    @pl.when(s == 0)
    def _():
        tail = NS * OUT_ROWS
        pltpu.sync_copy(agg_sh.at[pl.ds(tail, N - tail)],
                        out_hbm.at[c, pl.ds(tail, N - tail)])


def _sc_aggregate(x, packed, zeros):
    mesh = plsc.VectorSubcoreMesh(core_axis_name="c", subcore_axis_name="s")
    f = pl.kernel(
        _sc_agg_body,
        out_type=jax.ShapeDtypeStruct((NC, N, D), jnp.float32),
        mesh=mesh,
        scratch_types=[
            pltpu.VMEM((NCH // 2, 128), jnp.int32),
            pltpu.VMEM((4, W), jnp.int32),
            pltpu.VMEM((4, W), jnp.int32),
            pltpu.VMEM((W, D), jnp.float32),
            pltpu.VMEM((W, D), jnp.float32),
            pltpu.VMEM((W, D), jnp.float32),
            pltpu.VMEM((W, D), jnp.float32),
            pltpu.VMEM_SHARED((PAD_N, D), jnp.float32),
        ] + [pltpu.SemaphoreType.DMA] * 8,
    )
    return f(x, packed, zeros)


def _mlp_body(x_ref, agg_ref, w1_ref, b1_ref, w2_ref, b2_ref, eps_ref, o_ref):
    out = (x_ref[...] * (1.0 + eps_ref[0])
           + agg_ref[0] + agg_ref[1])
    h = jnp.dot(out, w1_ref[...], preferred_element_type=jnp.float32)
    h = jnp.maximum(h + b1_ref[...], 0.0)
    o_ref[...] = (jnp.dot(h, w2_ref[...], preferred_element_type=jnp.float32)
                  + b2_ref[...])


def _mlp(x, agg, W1, b1, W2, b2, eps):
    R = 1000
    grid = (N // R,)
    return pl.pallas_call(
        _mlp_body,
        grid=grid,
        in_specs=[
            pl.BlockSpec((R, D), lambda i: (i, 0)),
            pl.BlockSpec((NC, R, D), lambda i: (0, i, 0)),
            pl.BlockSpec((D, D), lambda i: (0, 0)),
            pl.BlockSpec((1, D), lambda i: (0, 0)),
            pl.BlockSpec((D, D), lambda i: (0, 0)),
            pl.BlockSpec((1, D), lambda i: (0, 0)),
            pl.BlockSpec(memory_space=pltpu.SMEM),
        ],
        out_specs=pl.BlockSpec((R, D), lambda i: (i, 0)),
        out_shape=jax.ShapeDtypeStruct((N, D), jnp.float32),
    )(x, agg, W1, b1.reshape(1, D), W2, b2.reshape(1, D), eps)


def kernel(x, edge_index, W1, b1, W2, b2, eps):
    row = edge_index[0]
    col = edge_index[1]
    npad = E_PAD - E
    ar = jnp.arange(npad, dtype=jnp.int32)
    pad_row = N + ar % (PAD_N - N)
    pad_col = ar % N
    row_p = jnp.concatenate([row, pad_row])
    col_p = jnp.concatenate([col, pad_col])
    packed = ((row_p << SHIFT) | col_p).reshape(E_PAD // 128, 128)
    zeros = jnp.zeros((STRIPE, D), jnp.float32)
    agg = _sc_aggregate(x, packed, zeros)
    return _mlp(x, agg, W1, b1, W2, b2, eps)


# speedup vs baseline: 13.8114x; 1.0043x over previous
import jax
import jax.numpy as jnp
from jax import lax
from jax.experimental import pallas as pl
from jax.experimental.pallas import tpu as pltpu
from jax.experimental.pallas import tpu_sc as plsc

N = 10000
D = 128
E = 320000

NC = 2
NS = 16
W = 32
NCH = 320
E_PAD = NC * NS * NCH * W
PAD_N = 10112
STRIPE = PAD_N // NS
OUT_ROWS = 624
SHIFT = 14
MASK = (1 << SHIFT) - 1
NB = 8
HW = NB // 2
PK_PER_ROW = 128 // W


def _sc_agg_body(x_hbm, pk_hbm, z_hbm, out_hbm,
                 pk_v, colr, rowr, gb, agg_sh, *sems):
    c = lax.axis_index("c")
    s = lax.axis_index("s")
    gsems = sems[:NB]
    ssems = sems[NB:]

    pltpu.sync_copy(z_hbm, agg_sh.at[pl.ds(s * STRIPE, STRIPE)])

    base = (c * NS + s) * (NCH // PK_PER_ROW)
    pltpu.sync_copy(pk_hbm.at[pl.ds(base, NCH // PK_PER_ROW)], pk_v)

    def unpack(m, ring):
        @pl.loop(0, W, step=16)
        def _(k):
            p = pk_v[m // PK_PER_ROW, pl.ds((m % PK_PER_ROW) * W + k, 16)]
            cc = p & jnp.int32(MASK)
            r = jax.lax.shift_right_logical(p, SHIFT)
            colr[ring, pl.ds(k, 16)] = cc
            rowr[ring, pl.ds(k, 16)] = jnp.where(
                r == cc, jnp.int32(N) + (r & jnp.int32(63)), r)

    def start_gather(j, b):
        pltpu.async_copy(x_hbm.at[colr.at[b]], gb.at[b], gsems[b])

    def wait_gather(j, b):
        pltpu.make_async_copy(x_hbm.at[colr.at[b]], gb.at[b],
                              gsems[b]).wait()

    def start_scatter(j, b):
        pltpu.async_copy(gb.at[b], agg_sh.at[rowr.at[b]], ssems[b],
                         add=True)

    def wait_scatter(j, b):
        pltpu.make_async_copy(gb.at[b], agg_sh.at[rowr.at[b]],
                              ssems[b]).wait()

    for m in range(HW):
        unpack(m, m)

    plsc.subcore_barrier()

    for m in range(HW):
        start_gather(m, m)
    for m in range(HW):
        wait_gather(m, m)
        start_scatter(m, m)
        unpack(m + HW, m + HW)
        start_gather(m + HW, m + HW)

    @pl.loop(HW, NCH - HW - (NB - 1), step=NB)
    def _(j):
        for k in range(NB):
            wait_scatter(j + k - HW, k)
            unpack(j + k + HW, k)
            start_gather(j + k + HW, k)
            wait_gather(j + k, (HW + k) % NB)
            start_scatter(j + k, (HW + k) % NB)

    for m in range(NCH - HW, NCH):
        wait_gather(m, m % NB)
        start_scatter(m, m % NB)
    for m in range(NCH - NB, NCH):
        wait_scatter(m, m % NB)

    plsc.subcore_barrier()
    pltpu.sync_copy(agg_sh.at[pl.ds(s * OUT_ROWS, OUT_ROWS)],
                    out_hbm.at[c, pl.ds(s * OUT_ROWS, OUT_ROWS)])

    @pl.when(s == 0)
    def _():
        tail = NS * OUT_ROWS
        pltpu.sync_copy(agg_sh.at[pl.ds(tail, N - tail)],
                        out_hbm.at[c, pl.ds(tail, N - tail)])


def _sc_aggregate(x, packed, zeros):
    mesh = plsc.VectorSubcoreMesh(core_axis_name="c", subcore_axis_name="s")
    f = pl.kernel(
        _sc_agg_body,
        out_type=jax.ShapeDtypeStruct((NC, N, D), jnp.float32),
        mesh=mesh,
        scratch_types=[
            pltpu.VMEM((NCH // PK_PER_ROW, 128), jnp.int32),
            pltpu.VMEM((NB, W), jnp.int32),
            pltpu.VMEM((NB, W), jnp.int32),
            pltpu.VMEM((NB, W, D), jnp.float32),
            pltpu.VMEM_SHARED((PAD_N, D), jnp.float32),
        ] + [pltpu.SemaphoreType.DMA] * (2 * NB),
    )
    return f(x, packed, zeros)


def _mlp_body(x_ref, agg_ref, w1_ref, b1_ref, w2_ref, b2_ref, eps_ref, o_ref):
    out = (x_ref[...] * (1.0 + eps_ref[0])
           + agg_ref[0] + agg_ref[1])
    h = jnp.dot(out, w1_ref[...], preferred_element_type=jnp.float32)
    h = jnp.maximum(h + b1_ref[...], 0.0)
    o_ref[...] = (jnp.dot(h, w2_ref[...], preferred_element_type=jnp.float32)
                  + b2_ref[...])


def _mlp(x, agg, W1, b1, W2, b2, eps):
    R = 1000
    grid = (N // R,)
    return pl.pallas_call(
        _mlp_body,
        grid=grid,
        in_specs=[
            pl.BlockSpec((R, D), lambda i: (i, 0)),
            pl.BlockSpec((NC, R, D), lambda i: (0, i, 0)),
            pl.BlockSpec((D, D), lambda i: (0, 0)),
            pl.BlockSpec((1, D), lambda i: (0, 0)),
            pl.BlockSpec((D, D), lambda i: (0, 0)),
            pl.BlockSpec((1, D), lambda i: (0, 0)),
            pl.BlockSpec(memory_space=pltpu.SMEM),
        ],
        out_specs=pl.BlockSpec((R, D), lambda i: (i, 0)),
        out_shape=jax.ShapeDtypeStruct((N, D), jnp.float32),
    )(x, agg, W1, b1.reshape(1, D), W2, b2.reshape(1, D), eps)


def kernel(x, edge_index, W1, b1, W2, b2, eps):
    row = edge_index[0]
    col = edge_index[1]
    npad = E_PAD - E
    ar = jnp.arange(npad, dtype=jnp.int32)
    pad_row = N + ar % (PAD_N - N)
    pad_col = ar % N
    row_p = jnp.concatenate([row, pad_row])
    col_p = jnp.concatenate([col, pad_col])
    packed = ((row_p << SHIFT) | col_p).reshape(E_PAD // 128, 128)
    zeros = jnp.zeros((STRIPE, D), jnp.float32)
    agg = _sc_aggregate(x, packed, zeros)
    return _mlp(x, agg, W1, b1, W2, b2, eps)
